# baseline (speedup 1.0000x reference)
import jax
import jax.numpy as jnp
from jax.experimental import pallas as pl
from jax.experimental.pallas import tpu as pltpu

_EPS = 0.05
_ITERS = 5
_N = 1024
_TINY = 1e-30


def _loss_body(p_ref, tT_ref, out_ref):
    inv_eps = jnp.float32(1.0 / _EPS)
    p = p_ref[0]
    t_t = tT_ref[0]
    pn20 = jnp.sum(p * p, axis=1, keepdims=True) * inv_eps
    tn20 = jnp.sum(t_t * t_t, axis=0, keepdims=True) * inv_eps
    cross = jax.lax.dot_general(
        p, t_t, (((1,), (0,)), ((), ())),
        preferred_element_type=jnp.float32)
    c20 = (pn20 + tn20) - (2.0 * inv_eps) * cross

    d1 = jnp.min(c20, axis=1, keepdims=True)
    d2 = jnp.min(c20, axis=0, keepdims=True)
    m = c20 - d1
    s2 = jnp.min(m, axis=0, keepdims=True)
    cd = (jnp.sum(d1) + 0.5 * jnp.sum(d2)) * jnp.float32(_EPS / _N)

    inv_n = jnp.float32(1.0 / _N)
    e = jnp.exp(s2 - m)
    w = jnp.exp(-s2)
    v = jnp.zeros((_N, 1), dtype=jnp.float32)
    for _ in range(_ITERS):
        s = jnp.maximum(jnp.sum(e * w, axis=1, keepdims=True), _TINY)
        v = inv_n / s
        t_sum = jnp.maximum(jnp.sum(e * v, axis=0, keepdims=True), _TINY)
        w = inv_n / t_sum

    emd = jnp.sum(v * jnp.sum((e * c20) * w, axis=1, keepdims=True))
    emd = emd * jnp.float32(_EPS * _N)
    out_ref[...] = jnp.full((1, 1, 128), cd + emd, dtype=jnp.float32)


def kernel(target, pre):
    bsz = pre.shape[0]
    t_t = jnp.swapaxes(target, 1, 2)
    per_batch = pl.pallas_call(
        _loss_body,
        grid=(bsz,),
        in_specs=[
            pl.BlockSpec((1, _N, 3), lambda b: (b, 0, 0)),
            pl.BlockSpec((1, 3, _N), lambda b: (b, 0, 0)),
        ],
        out_specs=pl.BlockSpec((1, 1, 128), lambda b: (b, 0, 0)),
        out_shape=jax.ShapeDtypeStruct((bsz, 1, 128), jnp.float32),
    )(pre, t_t)
    return jnp.sum(per_batch[:, 0, 0])


# speedup vs baseline: 1.0253x; 1.0253x over previous
import jax
import jax.numpy as jnp
from jax.experimental import pallas as pl
from jax.experimental.pallas import tpu as pltpu

_EPS = 0.05
_ITERS = 5
_N = 1024
_TINY = 1e-30


def _loss_body(p_ref, tT_ref, out_ref):
    inv_eps = jnp.float32(1.0 / _EPS)
    p = p_ref[0]
    t_t = tT_ref[0]
    pn20 = jnp.sum(p * p, axis=1, keepdims=True) * inv_eps
    tn20 = jnp.sum(t_t * t_t, axis=0, keepdims=True) * inv_eps
    cross = jax.lax.dot_general(
        p, t_t, (((1,), (0,)), ((), ())),
        preferred_element_type=jnp.float32)
    c20 = (pn20 + tn20) - (2.0 * inv_eps) * cross

    d1 = jnp.min(c20, axis=1, keepdims=True)
    d2 = jnp.min(c20, axis=0, keepdims=True)
    m = c20 - d1
    s2 = jnp.min(m, axis=0, keepdims=True)
    cd = (jnp.sum(d1) + 0.5 * jnp.sum(d2)) * jnp.float32(_EPS / _N)

    inv_n = jnp.float32(1.0 / _N)
    e = jnp.exp(s2 - m)
    w = jnp.exp(-s2)
    v = jnp.zeros((_N, 1), dtype=jnp.float32)
    for _ in range(_ITERS):
        s = jnp.maximum(jnp.sum(e * w, axis=1, keepdims=True), _TINY)
        v = inv_n / s
        t_sum = jnp.maximum(jnp.sum(e * v, axis=0, keepdims=True), _TINY)
        w = inv_n / t_sum

    emd = jnp.sum(v * jnp.sum((e * c20) * w, axis=1, keepdims=True))
    emd = emd * jnp.float32(_EPS * _N)
    loss = jnp.full((1, 1, 128), cd + emd, dtype=jnp.float32)

    @pl.when(pl.program_id(0) == 0)
    def _init():
        out_ref[...] = loss

    @pl.when(pl.program_id(0) != 0)
    def _acc():
        out_ref[...] += loss


def kernel(target, pre):
    bsz = pre.shape[0]
    t_t = jnp.swapaxes(target, 1, 2)
    total = pl.pallas_call(
        _loss_body,
        grid=(bsz,),
        in_specs=[
            pl.BlockSpec((1, _N, 3), lambda b: (b, 0, 0)),
            pl.BlockSpec((1, 3, _N), lambda b: (b, 0, 0)),
        ],
        out_specs=pl.BlockSpec((1, 1, 128), lambda b: (0, 0, 0)),
        out_shape=jax.ShapeDtypeStruct((1, 1, 128), jnp.float32),
    )(pre, t_t)
    return total[0, 0, 0]
